# baseline (speedup 1.0000x reference)
import jax
import jax.numpy as jnp
from jax.experimental import pallas as pl

_N_E = 1024
_E_DIM = 256
_BETA = 0.25
_N_TOK = 8 * 32 * 32
_BLOCK = 4096
_N_BLK = _N_TOK // _BLOCK


def _vq_kernel(z_ref, emb_ref, zq_ref, idx_ref, loss_ref, counts_ref, perp_ref):
    i = pl.program_id(0)
    z = z_ref[...]
    emb = emb_ref[...]
    dots = jax.lax.dot_general(z, emb, (((1,), (1,)), ((), ())),
                               preferred_element_type=jnp.float32)
    row2 = jnp.sum(z * z, axis=1, keepdims=True)
    e2 = jnp.sum(emb * emb, axis=1)
    d = (row2 + e2[None, :]) - 2.0 * dots
    dmin = jnp.min(d, axis=1, keepdims=True)
    cols = jax.lax.broadcasted_iota(jnp.int32, d.shape, 1)
    idx = jnp.min(jnp.where(d <= dmin, cols, _N_E), axis=1, keepdims=True)
    onehot = (cols == idx).astype(jnp.float32)
    zq = jax.lax.dot_general(onehot, emb, (((1,), (0,)), ((), ())),
                             preferred_element_type=jnp.float32)
    zq_ref[...] = zq
    idx_ref[...] = idx

    sse_part = jnp.sum(jnp.maximum(dmin, 0.0))
    ones_row = jnp.ones((1, _BLOCK), jnp.float32)
    counts_part = jax.lax.dot_general(ones_row, onehot, (((1,), (0,)), ((), ())),
                                      preferred_element_type=jnp.float32)

    @pl.when(i == 0)
    def _init():
        loss_ref[...] = jnp.zeros_like(loss_ref)
        counts_ref[...] = jnp.zeros_like(counts_ref)
        perp_ref[...] = jnp.zeros_like(perp_ref)

    loss_ref[...] += jnp.full((1, 1), sse_part, jnp.float32)
    counts_ref[...] += counts_part

    @pl.when(i == _N_BLK - 1)
    def _finish():
        probs = counts_ref[...] / _N_TOK
        ent = jnp.sum(probs * jnp.log(probs + 1e-10))
        perp_ref[...] = jnp.full((1, 1), jnp.exp(-ent), jnp.float32)
        loss_ref[...] = loss_ref[...] * ((1.0 + _BETA) / (_N_TOK * _E_DIM))


def kernel(inputs, embedding):
    b, c, h, w = inputs.shape
    z = jnp.transpose(inputs, (0, 2, 3, 1)).reshape(-1, c)
    zq, idx, loss, _counts, perp = pl.pallas_call(
        _vq_kernel,
        grid=(_N_BLK,),
        in_specs=[
            pl.BlockSpec((_BLOCK, _E_DIM), lambda i: (i, 0)),
            pl.BlockSpec((_N_E, _E_DIM), lambda i: (0, 0)),
        ],
        out_specs=[
            pl.BlockSpec((_BLOCK, _E_DIM), lambda i: (i, 0)),
            pl.BlockSpec((_BLOCK, 1), lambda i: (i, 0)),
            pl.BlockSpec((1, 1), lambda i: (0, 0)),
            pl.BlockSpec((1, _N_E), lambda i: (0, 0)),
            pl.BlockSpec((1, 1), lambda i: (0, 0)),
        ],
        out_shape=[
            jax.ShapeDtypeStruct((_N_TOK, _E_DIM), jnp.float32),
            jax.ShapeDtypeStruct((_N_TOK, 1), jnp.int32),
            jax.ShapeDtypeStruct((1, 1), jnp.float32),
            jax.ShapeDtypeStruct((1, _N_E), jnp.float32),
            jax.ShapeDtypeStruct((1, 1), jnp.float32),
        ],
    )(z, embedding)
    quantized = jnp.transpose(zq.reshape(b, h, w, c), (0, 3, 1, 2))
    encoding_indices = idx.reshape(b, h, w)
    return (loss[0, 0], quantized, perp[0, 0], encoding_indices)


# speedup vs baseline: 1.1220x; 1.1220x over previous
import jax
import jax.numpy as jnp
from jax.experimental import pallas as pl

_N_E = 1024
_E_DIM = 256
_BETA = 0.25
_N_TOK = 8 * 32 * 32
_BLOCK = 2048
_N_BLK = _N_TOK // _BLOCK


def _vq_kernel(z_ref, emb_ref, zq_ref, idx_ref, loss_ref, counts_ref, perp_ref):
    i = pl.program_id(0)
    z = z_ref[...]
    emb = emb_ref[...]
    m2dots = jax.lax.dot_general(z, emb * -2.0, (((1,), (1,)), ((), ())),
                                 preferred_element_type=jnp.float32)
    row2 = jnp.sum(z * z, axis=1, keepdims=True)
    e2 = jnp.sum(emb * emb, axis=1)
    _C = 128
    _N_CHUNK = _N_E // _C
    state_v = (row2 + e2[None, :_C]) + m2dots[:, :_C]
    state_c = jnp.zeros((_BLOCK, _C), jnp.int32)
    for c in range(1, _N_CHUNK):
        d_c = (row2 + e2[None, c * _C:(c + 1) * _C]) + m2dots[:, c * _C:(c + 1) * _C]
        lt = d_c < state_v
        state_v = jnp.minimum(state_v, d_c)
        state_c = jnp.where(lt, c, state_c)
    dmin = jnp.min(state_v, axis=1, keepdims=True)
    lanes = jax.lax.broadcasted_iota(jnp.int32, (_BLOCK, _C), 1)
    gidx = state_c * _C + lanes
    idx = jnp.min(jnp.where(state_v <= dmin, gidx, _N_E), axis=1, keepdims=True)
    cols = jax.lax.broadcasted_iota(jnp.int32, (_BLOCK, _N_E), 1)
    onehot = (cols == idx).astype(jnp.float32)
    zq = jax.lax.dot_general(onehot, emb, (((1,), (0,)), ((), ())),
                             preferred_element_type=jnp.float32)
    zq_ref[...] = zq
    idx_ref[...] = idx

    sse_part = jnp.sum(jnp.maximum(dmin, 0.0))
    ones_row = jnp.ones((1, _BLOCK), jnp.float32)
    counts_part = jax.lax.dot_general(ones_row, onehot, (((1,), (0,)), ((), ())),
                                      preferred_element_type=jnp.float32)

    @pl.when(i == 0)
    def _init():
        loss_ref[...] = jnp.zeros_like(loss_ref)
        counts_ref[...] = jnp.zeros_like(counts_ref)
        perp_ref[...] = jnp.zeros_like(perp_ref)

    loss_ref[...] += jnp.full((1, 1), sse_part, jnp.float32)
    counts_ref[...] += counts_part

    @pl.when(i == _N_BLK - 1)
    def _finish():
        probs = counts_ref[...] / _N_TOK
        ent = jnp.sum(probs * jnp.log(probs + 1e-10))
        perp_ref[...] = jnp.full((1, 1), jnp.exp(-ent), jnp.float32)
        loss_ref[...] = loss_ref[...] * ((1.0 + _BETA) / (_N_TOK * _E_DIM))


def kernel(inputs, embedding):
    b, c, h, w = inputs.shape
    z = jnp.transpose(inputs, (0, 2, 3, 1)).reshape(-1, c)
    zq, idx, loss, _counts, perp = pl.pallas_call(
        _vq_kernel,
        grid=(_N_BLK,),
        in_specs=[
            pl.BlockSpec((_BLOCK, _E_DIM), lambda i: (i, 0)),
            pl.BlockSpec((_N_E, _E_DIM), lambda i: (0, 0)),
        ],
        out_specs=[
            pl.BlockSpec((_BLOCK, _E_DIM), lambda i: (i, 0)),
            pl.BlockSpec((_BLOCK, 1), lambda i: (i, 0)),
            pl.BlockSpec((1, 1), lambda i: (0, 0)),
            pl.BlockSpec((1, _N_E), lambda i: (0, 0)),
            pl.BlockSpec((1, 1), lambda i: (0, 0)),
        ],
        out_shape=[
            jax.ShapeDtypeStruct((_N_TOK, _E_DIM), jnp.float32),
            jax.ShapeDtypeStruct((_N_TOK, 1), jnp.int32),
            jax.ShapeDtypeStruct((1, 1), jnp.float32),
            jax.ShapeDtypeStruct((1, _N_E), jnp.float32),
            jax.ShapeDtypeStruct((1, 1), jnp.float32),
        ],
    )(z, embedding)
    quantized = jnp.transpose(zq.reshape(b, h, w, c), (0, 3, 1, 2))
    encoding_indices = idx.reshape(b, h, w)
    return (loss[0, 0], quantized, perp[0, 0], encoding_indices)
